# baseline (speedup 1.0000x reference)
import jax
import jax.numpy as jnp
from jax.experimental import pallas as pl
from jax.experimental.pallas import tpu as pltpu

_N_NODES = 10000
_D_FEAT = 128
_DROP_IDX = _N_NODES // 2
_BLOCK = 10000
_N_BLOCKS = _N_NODES // _BLOCK
_DROP_BLOCK = _DROP_IDX // _BLOCK
_DROP_OFF = _DROP_IDX % _BLOCK


def _body(x_ref, o_ref):
    o_ref[...] = x_ref[...]

    @pl.when(pl.program_id(0) == _DROP_BLOCK)
    def _():
        o_ref[pl.ds(_DROP_OFF, 1), :] = jnp.zeros((1, _D_FEAT), jnp.float32)


@jax.jit
def _drop_row_copy(x):
    return pl.pallas_call(
        _body,
        grid=(_N_BLOCKS,),
        in_specs=[
            pl.BlockSpec((_BLOCK, _D_FEAT), lambda i: (i, 0)),
        ],
        out_specs=pl.BlockSpec((_BLOCK, _D_FEAT), lambda i: (i, 0)),
        out_shape=jax.ShapeDtypeStruct((_N_NODES, _D_FEAT), jnp.float32),
        compiler_params=pltpu.CompilerParams(
            dimension_semantics=("arbitrary",),
        ),
    )(x)


def kernel(x, adj):
    return (_drop_row_copy(x), adj)
